# Initial kernel scaffold; baseline (speedup 1.0000x reference)
import jax
import jax.numpy as jnp
from jax.experimental import pallas as pl


def kernel(inputs, emb_table):
    raise NotImplementedError("write your pallas kernel here")


# speedup vs baseline: 1.4607x; 1.4607x over previous
import functools

import jax
import jax.numpy as jnp
from jax import lax
from jax.experimental import pallas as pl
from jax.experimental.pallas import tpu as pltpu
from jax.experimental.pallas import tpu_sc as plsc

NC = 2
NS = 16
NW = NC * NS

G = 128
K = 8
CHUNK = K * G


@functools.partial(jax.jit, static_argnums=(2, 3))
def _emb_lookup(idx2d, table, n_chunks, emb_dim):
    b_total = idx2d.shape[0] * G
    mesh = plsc.VectorSubcoreMesh(core_axis_name="c", subcore_axis_name="s")

    @functools.partial(
        pl.kernel,
        out_type=jax.ShapeDtypeStruct((b_total, emb_dim), jnp.float32),
        mesh=mesh,
        scratch_types=[
            pltpu.VMEM((K, G), jnp.int32),
            pltpu.VMEM((CHUNK, emb_dim), jnp.float32),
            pltpu.SemaphoreType.DMA,
        ],
        compiler_params=pltpu.CompilerParams(use_tc_tiling_on_sc=False),
    )
    def body(idx_hbm, table_hbm, out_hbm, idx_v, rows_v, sem):
        wid = lax.axis_index("s") * NC + lax.axis_index("c")
        row0 = wid * (n_chunks * K)

        def chunk_body(i, carry):
            pltpu.sync_copy(idx_hbm.at[pl.ds(row0 + i * K, K)], idx_v)
            copies = [
                pltpu.async_copy(
                    table_hbm.at[idx_v.at[j]],
                    rows_v.at[pl.ds(j * G, G)],
                    sem,
                )
                for j in range(K)
            ]
            for c in copies:
                c.wait()
            pltpu.sync_copy(
                rows_v, out_hbm.at[pl.ds((row0 + i * K) * G, CHUNK)]
            )
            return carry

        lax.fori_loop(0, n_chunks, chunk_body, 0)

    return body(idx2d, table)


def kernel(inputs, emb_table):
    batch, hist = inputs.shape
    emb_dim = emb_table.shape[1]
    b_total = batch * hist
    assert b_total % (NW * CHUNK) == 0
    n_chunks = b_total // (NW * CHUNK)
    idx2d = inputs.reshape(-1, G).astype(jnp.int32)
    out = _emb_lookup(idx2d, emb_table, n_chunks, emb_dim)
    return out.reshape(batch, hist, emb_dim)


# speedup vs baseline: 1.4889x; 1.0193x over previous
import functools

import jax
import jax.numpy as jnp
from jax import lax
from jax.experimental import pallas as pl
from jax.experimental.pallas import tpu as pltpu
from jax.experimental.pallas import tpu_sc as plsc

NC = 2
NS = 16
NW = NC * NS

G = 128
K = 10
CHUNK = K * G


@functools.partial(jax.jit, static_argnums=(2, 3))
def _emb_lookup(idx2d, table, n_chunks, emb_dim):
    b_total = idx2d.shape[0] * G
    mesh = plsc.VectorSubcoreMesh(core_axis_name="c", subcore_axis_name="s")

    @functools.partial(
        pl.kernel,
        out_type=jax.ShapeDtypeStruct((b_total, emb_dim), jnp.float32),
        mesh=mesh,
        scratch_types=[
            pltpu.VMEM((K, G), jnp.int32),
            pltpu.VMEM((K, G), jnp.int32),
            pltpu.VMEM((CHUNK, emb_dim), jnp.float32),
            pltpu.VMEM((CHUNK, emb_dim), jnp.float32),
            pltpu.SemaphoreType.DMA,
            pltpu.SemaphoreType.DMA,
            pltpu.SemaphoreType.DMA,
        ],
        compiler_params=pltpu.CompilerParams(use_tc_tiling_on_sc=False),
    )
    def body(idx_hbm, table_hbm, out_hbm, idx_v0, idx_v1, rows_v0, rows_v1,
             gsem, osem0, osem1):
        wid = lax.axis_index("s") * NC + lax.axis_index("c")
        row0 = wid * (n_chunks * K)

        def do_chunk(i, idx_v, rows_v, osem, wait_out):
            pltpu.sync_copy(idx_hbm.at[pl.ds(row0 + i * K, K)], idx_v)
            if wait_out:
                pltpu.make_async_copy(
                    rows_v, out_hbm.at[pl.ds(0, CHUNK)], osem
                ).wait()
            copies = [
                pltpu.async_copy(
                    table_hbm.at[idx_v.at[j]],
                    rows_v.at[pl.ds(j * G, G)],
                    gsem,
                )
                for j in range(K)
            ]
            for c in copies:
                c.wait()
            pltpu.async_copy(
                rows_v, out_hbm.at[pl.ds((row0 + i * K) * G, CHUNK)], osem
            )

        do_chunk(0, idx_v0, rows_v0, osem0, False)
        do_chunk(1, idx_v1, rows_v1, osem1, False)

        def pair(j, carry):
            do_chunk(2 * j, idx_v0, rows_v0, osem0, True)
            do_chunk(2 * j + 1, idx_v1, rows_v1, osem1, True)
            return carry

        lax.fori_loop(1, n_chunks // 2, pair, 0)

        pltpu.make_async_copy(rows_v0, out_hbm.at[pl.ds(0, CHUNK)], osem0).wait()
        pltpu.make_async_copy(rows_v1, out_hbm.at[pl.ds(0, CHUNK)], osem1).wait()

    return body(idx2d, table)


def kernel(inputs, emb_table):
    batch, hist = inputs.shape
    emb_dim = emb_table.shape[1]
    b_total = batch * hist
    assert b_total % (NW * CHUNK) == 0 and (b_total // (NW * CHUNK)) % 2 == 0
    n_chunks = b_total // (NW * CHUNK)
    idx2d = inputs.reshape(-1, G).astype(jnp.int32)
    out = _emb_lookup(idx2d, emb_table, n_chunks, emb_dim)
    return out.reshape(batch, hist, emb_dim)
